# Initial kernel scaffold; baseline (speedup 1.0000x reference)
import jax
import jax.numpy as jnp
from jax.experimental import pallas as pl


def kernel(in_feat, edge_index, W1, b1, W2, b2):
    raise NotImplementedError("write your pallas kernel here")


# speedup vs baseline: 5.8644x; 5.8644x over previous
import functools

import jax
import jax.numpy as jnp
from jax import lax
from jax.experimental import pallas as pl
from jax.experimental.pallas import tpu as pltpu
from jax.experimental.pallas import tpu_sc as plsc

N = 10000
E = 320000
D = 128
H = 128

NC = 2
NS = 16
NW = NC * NS

NPAD = 10240
SLICE = NPAD // NS
B = 128
NCHUNK = E // B
FULL = NCHUNK // NW
EXTRA = NCHUNK - FULL * NW

_mesh = plsc.VectorSubcoreMesh(
    core_axis_name="c", subcore_axis_name="s", num_cores=NC, num_subcores=NS)


def _deg_kernel(src_hbm, dst_hbm, zeros_hbm, ones_hbm, out_hbm,
                idx_s, idx_d, ones_v, bins_o, bins_i, sem):
    cid = lax.axis_index("c")
    sid = lax.axis_index("s")
    wid = cid * NS + sid

    pltpu.sync_copy(zeros_hbm, bins_o.at[pl.ds(sid * SLICE, SLICE)])
    pltpu.sync_copy(zeros_hbm, bins_i.at[pl.ds(sid * SLICE, SLICE)])
    pltpu.sync_copy(ones_hbm, ones_v)
    plsc.subcore_barrier()

    def chunk_body(chunk):
        base = chunk * B
        pltpu.sync_copy(src_hbm.at[pl.ds(base, B)], idx_s)
        pltpu.sync_copy(dst_hbm.at[pl.ds(base, B)], idx_d)
        pltpu.sync_copy(ones_v, bins_o.at[idx_s], add=True)
        pltpu.sync_copy(ones_v, bins_i.at[idx_d], add=True)

    def loop_body(j, _):
        chunk_body(wid + NW * j)
        return _

    lax.fori_loop(0, FULL, loop_body, None)

    @pl.when(wid < EXTRA)
    def _():
        chunk_body(NW * FULL + wid)

    plsc.subcore_barrier()
    pltpu.sync_copy(bins_o.at[pl.ds(sid * SLICE, SLICE)],
                    out_hbm.at[cid, 0, pl.ds(sid * SLICE, SLICE)])
    pltpu.sync_copy(bins_i.at[pl.ds(sid * SLICE, SLICE)],
                    out_hbm.at[cid, 1, pl.ds(sid * SLICE, SLICE)])


def _agg_kernel(h_hbm, src_hbm, dst_hbm, zeros_hbm, out_hbm,
                idx_s, idx_d, rows, acc, sem):
    cid = lax.axis_index("c")
    sid = lax.axis_index("s")
    wid = cid * NS + sid

    pltpu.sync_copy(zeros_hbm, acc.at[pl.ds(sid * SLICE, SLICE)])
    plsc.subcore_barrier()

    def chunk_body(chunk):
        base = chunk * B
        pltpu.sync_copy(src_hbm.at[pl.ds(base, B)], idx_s)
        pltpu.sync_copy(dst_hbm.at[pl.ds(base, B)], idx_d)
        pltpu.async_copy(h_hbm.at[idx_s], rows, sem).wait()
        pltpu.sync_copy(rows, acc.at[idx_d], add=True)

    def loop_body(j, _):
        chunk_body(wid + NW * j)
        return _

    lax.fori_loop(0, FULL, loop_body, None)

    @pl.when(wid < EXTRA)
    def _():
        chunk_body(NW * FULL + wid)

    plsc.subcore_barrier()
    pltpu.sync_copy(acc.at[pl.ds(sid * SLICE, SLICE)],
                    out_hbm.at[cid, pl.ds(sid * SLICE, SLICE)])


def _sc_degrees(src, dst):
    zeros = jnp.zeros((SLICE,), jnp.float32)
    ones = jnp.ones((B,), jnp.float32)
    f = functools.partial(
        pl.kernel,
        out_type=jax.ShapeDtypeStruct((NC, 2, NPAD), jnp.float32),
        mesh=_mesh,
        scratch_types=[
            pltpu.VMEM((B,), jnp.int32),
            pltpu.VMEM((B,), jnp.int32),
            pltpu.VMEM((B,), jnp.float32),
            pltpu.VMEM_SHARED((NPAD,), jnp.float32),
            pltpu.VMEM_SHARED((NPAD,), jnp.float32),
            pltpu.SemaphoreType.DMA,
        ],
    )(_deg_kernel)
    return f(src, dst, zeros, ones)


def _sc_aggregate(h, src, dst):
    zeros = jnp.zeros((SLICE, H), jnp.float32)
    f = functools.partial(
        pl.kernel,
        out_type=jax.ShapeDtypeStruct((NC, NPAD, H), jnp.float32),
        mesh=_mesh,
        scratch_types=[
            pltpu.VMEM((B,), jnp.int32),
            pltpu.VMEM((B,), jnp.int32),
            pltpu.VMEM((B, H), jnp.float32),
            pltpu.VMEM_SHARED((NPAD, H), jnp.float32),
            pltpu.SemaphoreType.DMA,
        ],
    )(_agg_kernel)
    return f(h, src, dst, zeros)


ROWS_BLK = 1024
GRID = NPAD // ROWS_BLK


def _tc1_body(dp00, dp01, dp10, dp11, x, w, h, ns, nd):
    deg_o = dp00[...] + dp10[...]
    deg_i = dp01[...] + dp11[...]
    ns_v = lax.rsqrt(jnp.maximum(deg_o, 1.0))
    nd_v = lax.rsqrt(jnp.maximum(deg_i, 1.0))
    ns[...] = ns_v
    nd[...] = nd_v
    h[...] = jnp.dot(x[...] * ns_v, w[...], preferred_element_type=jnp.float32)


def _tc1(deg_parts, x_pad, w1):
    dp = [deg_parts[c, i].reshape(NPAD, 1) for c in range(NC) for i in range(2)]
    col = pl.BlockSpec((ROWS_BLK, 1), lambda i: (i, 0))
    mat = pl.BlockSpec((ROWS_BLK, D), lambda i: (i, 0))
    wsp = pl.BlockSpec((D, H), lambda i: (0, 0))
    return pl.pallas_call(
        _tc1_body,
        grid=(GRID,),
        in_specs=[col, col, col, col, mat, wsp],
        out_specs=[mat, col, col],
        out_shape=[
            jax.ShapeDtypeStruct((NPAD, H), jnp.float32),
            jax.ShapeDtypeStruct((NPAD, 1), jnp.float32),
            jax.ShapeDtypeStruct((NPAD, 1), jnp.float32),
        ],
    )(dp[0], dp[1], dp[2], dp[3], x_pad, w1)


def _tc2_body(a, ns, nd, b1, w, h):
    agg = a[0] + a[1]
    x2 = jnp.maximum(agg * nd[...] + b1[...], 0.0)
    h[...] = jnp.dot(x2 * ns[...], w[...], preferred_element_type=jnp.float32)


def _tc2(agg_parts, ns, nd, b1, w2):
    asp = pl.BlockSpec((NC, ROWS_BLK, H), lambda i: (0, i, 0))
    col = pl.BlockSpec((ROWS_BLK, 1), lambda i: (i, 0))
    bsp = pl.BlockSpec((1, H), lambda i: (0, 0))
    wsp = pl.BlockSpec((D, H), lambda i: (0, 0))
    mat = pl.BlockSpec((ROWS_BLK, H), lambda i: (i, 0))
    return pl.pallas_call(
        _tc2_body,
        grid=(GRID,),
        in_specs=[asp, col, col, bsp, wsp],
        out_specs=mat,
        out_shape=jax.ShapeDtypeStruct((NPAD, H), jnp.float32),
    )(agg_parts, ns, nd, b1.reshape(1, H), w2)


def _tc3_body(a, nd, b2, out):
    agg = a[0] + a[1]
    out[...] = agg * nd[...] + b2[...]


def _tc3(agg_parts, nd, b2):
    asp = pl.BlockSpec((NC, ROWS_BLK, H), lambda i: (0, i, 0))
    col = pl.BlockSpec((ROWS_BLK, 1), lambda i: (i, 0))
    bsp = pl.BlockSpec((1, H), lambda i: (0, 0))
    mat = pl.BlockSpec((ROWS_BLK, H), lambda i: (i, 0))
    return pl.pallas_call(
        _tc3_body,
        grid=(GRID,),
        in_specs=[asp, col, bsp],
        out_specs=mat,
        out_shape=jax.ShapeDtypeStruct((NPAD, H), jnp.float32),
    )(agg_parts, nd, b2.reshape(1, H))


def kernel(in_feat, edge_index, W1, b1, W2, b2):
    src = edge_index[0]
    dst = edge_index[1]

    deg_parts = _sc_degrees(src, dst)

    x_pad = jnp.zeros((NPAD, D), jnp.float32).at[:N].set(in_feat)
    h1, ns, nd = _tc1(deg_parts, x_pad, W1)

    agg1 = _sc_aggregate(h1, src, dst)
    h2 = _tc2(agg1, ns, nd, b1, W2)

    agg2 = _sc_aggregate(h2, src, dst)
    out = _tc3(agg2, nd, b2)
    return out[:N]
